# Initial kernel scaffold; baseline (speedup 1.0000x reference)
import jax
import jax.numpy as jnp
from jax.experimental import pallas as pl


def kernel(x, edge_index, overlap_similarity, overlap_length, W_enc, b_enc, WA, bA, WB, bB, WC, bC, WD, bD, WE, bE, bn_h_gamma, bn_h_beta, bn_e_gamma, bn_e_beta, W1, b1, W2, b2):
    raise NotImplementedError("write your pallas kernel here")


# speedup vs baseline: 3.0287x; 3.0287x over previous
import functools

import jax
import jax.numpy as jnp
from jax import lax
from jax.experimental import pallas as pl
from jax.experimental.pallas import tpu as pltpu
from jax.experimental.pallas import tpu_sc as plsc

N = 10000
E = 320000
D = 128
NC = 2
NS = 16
C = 80
BLKE = 400
NROW = E // C
EBLK = 4000
NBLK = 1000

@functools.lru_cache(maxsize=1)
def _mesh():
    return plsc.VectorSubcoreMesh(core_axis_name="c", subcore_axis_name="s",
                                  num_cores=NC, num_subcores=NS)


def _matmul_multi(x, Ws, bs):
    M = x.shape[0]
    nW = len(Ws)

    def body(x_ref, *refs):
        w_refs = refs[:nW]
        b_refs = refs[nW:2 * nW]
        o_refs = refs[2 * nW:]
        xv = x_ref[...]
        for w_r, b_r, o_r in zip(w_refs, b_refs, o_refs):
            o_r[...] = (jnp.dot(xv, w_r[...], preferred_element_type=jnp.float32)
                        + b_r[...])

    in_specs = ([pl.BlockSpec((NBLK, D), lambda i: (i, 0))]
                + [pl.BlockSpec((D, D), lambda i: (0, 0))] * nW
                + [pl.BlockSpec((1, D), lambda i: (0, 0))] * nW)
    out_specs = [pl.BlockSpec((NBLK, D), lambda i: (i, 0))] * nW
    out_shape = [jax.ShapeDtypeStruct((M, D), jnp.float32)] * nW
    return pl.pallas_call(
        body, grid=(M // NBLK,), in_specs=in_specs, out_specs=out_specs,
        out_shape=out_shape)(x, *Ws, *[b.reshape(1, D) for b in bs])


def _psum(v):
    while v.shape[0] > 1:
        h = v.shape[0] // 2
        r = v[:h] + v[h:2 * h]
        if v.shape[0] % 2:
            r = jnp.concatenate([r, v[2 * h:]], axis=0)
        v = r
    w = v.shape[1]
    while w > 1:
        v = v[:, :w // 2] + v[:, w // 2:w]
        w //= 2
    return v[0, 0]


def _ol_stats(ol2):
    def body_mu(x_ref, st_ref):
        s = _psum(x_ref[...])
        st_ref[...] = jnp.concatenate(
            [jnp.full((1, 128), s / E, jnp.float32),
             jnp.zeros((7, 128), jnp.float32)], axis=0)

    st1 = pl.pallas_call(
        body_mu,
        in_specs=[pl.BlockSpec(ol2.shape, lambda: (0, 0))],
        out_specs=pl.BlockSpec((8, 128), lambda: (0, 0)),
        out_shape=jax.ShapeDtypeStruct((8, 128), jnp.float32))(ol2)

    def body_var(x_ref, m_ref, st_ref):
        d = x_ref[...] - m_ref[0, 0]
        s2 = _psum(d * d)
        st_ref[...] = jnp.concatenate(
            [jnp.full((1, 128), m_ref[0, 0], jnp.float32),
             jnp.full((1, 128), s2 / E, jnp.float32),
             jnp.zeros((6, 128), jnp.float32)], axis=0)

    return pl.pallas_call(
        body_var,
        in_specs=[pl.BlockSpec(ol2.shape, lambda: (0, 0)),
                  pl.BlockSpec((8, 128), lambda: (0, 0))],
        out_specs=pl.BlockSpec((8, 128), lambda: (0, 0)),
        out_shape=jax.ShapeDtypeStruct((8, 128), jnp.float32))(ol2, st1)


def _encode(sim2, ol2, olst, W_enc, b_enc):
    w0 = W_enc[0].reshape(1, D)
    w1 = W_enc[1].reshape(1, D)

    def body(s_ref, o_ref, st_ref, w0_ref, w1_ref, b_ref, e_ref):
        mu = st_ref[0, 0]
        var = st_ref[1, 0]
        oln = (o_ref[...] - mu) / jnp.sqrt(var)
        x2 = jnp.concatenate([s_ref[...], oln], axis=1)
        w2 = jnp.concatenate([w0_ref[...], w1_ref[...]], axis=0)
        e_ref[...] = (jnp.dot(x2, w2, preferred_element_type=jnp.float32)
                      + b_ref[...])

    return pl.pallas_call(
        body, grid=(E // EBLK,),
        in_specs=[pl.BlockSpec((EBLK, 1), lambda i: (i, 0)),
                  pl.BlockSpec((EBLK, 1), lambda i: (i, 0)),
                  pl.BlockSpec((8, 128), lambda i: (0, 0)),
                  pl.BlockSpec((1, D), lambda i: (0, 0)),
                  pl.BlockSpec((1, D), lambda i: (0, 0)),
                  pl.BlockSpec((1, D), lambda i: (0, 0))],
        out_specs=pl.BlockSpec((EBLK, D), lambda i: (i, 0)),
        out_shape=jax.ShapeDtypeStruct((E, D), jnp.float32),
    )(sim2, ol2, olst, w0, w1, b_enc.reshape(1, D))


def _edge_hat(e, Ds, Ed, WC, bC):
    def body(e_ref, ds_ref, ed_ref, w_ref, b_ref, eh_ref, st_ref):
        ce = (jnp.dot(e_ref[...], w_ref[...], preferred_element_type=jnp.float32)
              + b_ref[...])
        eh = (ds_ref[...].astype(jnp.float32)
              + ed_ref[...].astype(jnp.float32)) + ce
        eh_ref[...] = eh

        @pl.when(pl.program_id(0) == 0)
        def _():
            st_ref[...] = jnp.zeros_like(st_ref)

        st_ref[0:1, :] += jnp.sum(eh, axis=0, keepdims=True)

    return pl.pallas_call(
        body, grid=(E // EBLK,),
        in_specs=[pl.BlockSpec((EBLK, D), lambda i: (i, 0)),
                  pl.BlockSpec((EBLK, D), lambda i: (i, 0)),
                  pl.BlockSpec((EBLK, D), lambda i: (i, 0)),
                  pl.BlockSpec((D, D), lambda i: (0, 0)),
                  pl.BlockSpec((1, D), lambda i: (0, 0))],
        out_specs=[pl.BlockSpec((EBLK, D), lambda i: (i, 0)),
                   pl.BlockSpec((8, 128), lambda i: (0, 0))],
        out_shape=[jax.ShapeDtypeStruct((E, D), jnp.float32),
                   jax.ShapeDtypeStruct((8, 128), jnp.float32)],
    )(e, Ds, Ed, WC, bC.reshape(1, D))


def _edge_var(e_hat, stats):
    def body(eh_ref, s_ref, st_ref):
        mu = s_ref[0:1, :] / E

        @pl.when(pl.program_id(0) == 0)
        def _():
            st_ref[...] = jnp.zeros_like(st_ref)
            st_ref[0:1, :] += s_ref[0:1, :]

        d = eh_ref[...] - mu
        st_ref[1:2, :] += jnp.sum(d * d, axis=0, keepdims=True)

    return pl.pallas_call(
        body, grid=(E // EBLK,),
        in_specs=[pl.BlockSpec((EBLK, D), lambda i: (i, 0)),
                  pl.BlockSpec((8, 128), lambda i: (0, 0))],
        out_specs=pl.BlockSpec((8, 128), lambda i: (0, 0)),
        out_shape=jax.ShapeDtypeStruct((8, 128), jnp.float32),
    )(e_hat, stats)


def _edge_update(e, e_hat, stats, Bs, gamma, beta):
    def body(e_ref, eh_ref, st_ref, bs_ref, g_ref, b_ref,
             en_ref, sg_ref, mg_ref):
        mu = st_ref[0:1, :] / E
        var = st_ref[1:2, :] / E
        bn = (g_ref[...] * (eh_ref[...] - mu)) / jnp.sqrt(var + 1e-5) + b_ref[...]
        en = e_ref[...] + jnp.maximum(bn, 0.0)
        sg = jax.nn.sigmoid(en)
        en_ref[...] = en
        sg_ref[...] = sg
        mg_ref[...] = sg * bs_ref[...].astype(jnp.float32)

    return pl.pallas_call(
        body, grid=(E // EBLK,),
        in_specs=[pl.BlockSpec((EBLK, D), lambda i: (i, 0)),
                  pl.BlockSpec((EBLK, D), lambda i: (i, 0)),
                  pl.BlockSpec((8, 128), lambda i: (0, 0)),
                  pl.BlockSpec((EBLK, D), lambda i: (i, 0)),
                  pl.BlockSpec((1, D), lambda i: (0, 0)),
                  pl.BlockSpec((1, D), lambda i: (0, 0))],
        out_specs=[pl.BlockSpec((EBLK, D), lambda i: (i, 0))] * 3,
        out_shape=[jax.ShapeDtypeStruct((E, D), jnp.float32)] * 3,
    )(e, e_hat, stats, Bs, gamma.reshape(1, D), beta.reshape(1, D))


def _node_update(h, Ah, num, den, gamma, beta):
    def body(h_ref, a_ref, n_ref, d_ref, g_ref, b_ref, o_ref):
        hh = a_ref[...] + n_ref[...] / (d_ref[...] + 1e-6)
        mu = jnp.mean(hh, axis=0, keepdims=True)
        var = jnp.mean((hh - mu) ** 2, axis=0, keepdims=True)
        bn = (g_ref[...] * (hh - mu)) / jnp.sqrt(var + 1e-5) + b_ref[...]
        o_ref[...] = h_ref[...] + jnp.maximum(bn, 0.0)

    full = pl.BlockSpec((N, D), lambda: (0, 0))
    row = pl.BlockSpec((1, D), lambda: (0, 0))
    return pl.pallas_call(
        body,
        in_specs=[full, full, full, full, row, row],
        out_specs=full,
        out_shape=jax.ShapeDtypeStruct((N, D), jnp.float32),
    )(h, Ah, num, den, gamma.reshape(1, D), beta.reshape(1, D))


def _decode(Hs, Hd, e, W1c, b1, W2, b2):
    w2r = W2.reshape(1, D)

    def body(hs_ref, hd_ref, e_ref, w_ref, b_ref, w2_ref, b2_ref, p_ref):
        t = (hs_ref[...].astype(jnp.float32) + hd_ref[...].astype(jnp.float32)
             + jnp.dot(e_ref[...], w_ref[...], preferred_element_type=jnp.float32)
             + b_ref[...])
        t = jnp.maximum(t, 0.0)
        p_ref[...] = (jnp.sum(t * w2_ref[...], axis=-1, keepdims=True)
                      + b2_ref[...])

    return pl.pallas_call(
        body, grid=(E // EBLK,),
        in_specs=[pl.BlockSpec((EBLK, D), lambda i: (i, 0)),
                  pl.BlockSpec((EBLK, D), lambda i: (i, 0)),
                  pl.BlockSpec((EBLK, D), lambda i: (i, 0)),
                  pl.BlockSpec((D, D), lambda i: (0, 0)),
                  pl.BlockSpec((1, D), lambda i: (0, 0)),
                  pl.BlockSpec((1, D), lambda i: (0, 0)),
                  pl.BlockSpec((1, 1), lambda i: (0, 0))],
        out_specs=pl.BlockSpec((EBLK, 1), lambda i: (i, 0)),
        out_shape=jax.ShapeDtypeStruct((E, 1), jnp.float32),
    )(Hs, Hd, e, W1c, b1.reshape(1, D), w2r, b2.reshape(1, 1))


def _sc_gather(tab_src, tab_dst, tab_b, src_r, dst_r):
    nb = 0 if tab_b is None else 1
    out_type = [jax.ShapeDtypeStruct((E, D), jnp.float32)] * (2 + nb)
    scratch = [pltpu.VMEM((BLKE,), jnp.int32),
               pltpu.VMEM((BLKE, D), jnp.float32),
               pltpu.VMEM((BLKE, D), jnp.float32),
               pltpu.SemaphoreType.DMA,
               pltpu.SemaphoreType.DMA,
               pltpu.SemaphoreType.DMA,
               pltpu.SemaphoreType.DMA]
    tabs = (tab_src, tab_dst) + ((tab_b,) if nb else ())

    @functools.partial(pl.kernel, out_type=out_type, mesh=_mesh(),
                       scratch_types=scratch)
    def k(s_hbm, d_hbm, *refs):
        t_hbms = refs[:2 + nb]
        o_hbms = refs[2 + nb:4 + 2 * nb]
        ibuf, buf_a, buf_b, gsa, gsb, wsa, wsb = refs[4 + 2 * nb:]
        cid = lax.axis_index("c")
        sid = lax.axis_index("s")

        def run(tab, idx_r, out, base_edges, n_edges, buf, gsem, wsem):
            nblk = n_edges // BLKE

            @pl.loop(0, nblk)
            def _(g):
                eb = base_edges + g * BLKE
                pltpu.sync_copy(idx_r.at[pl.ds(eb, BLKE)], ibuf)
                descs = [pltpu.async_copy(tab.at[ibuf.at[pl.ds(kk * C, C)]],
                                          buf.at[pl.ds(kk * C, C)], gsem)
                         for kk in range(5)]
                for dsc in descs:
                    dsc.wait()
                pltpu.async_copy(buf, out.at[pl.ds(eb, BLKE)], wsem).wait()

        @pl.when(cid == 0)
        def _():
            run(t_hbms[0], s_hbm, o_hbms[0], sid * (E // NS), E // NS,
                buf_a, gsa, wsa)
            if nb:
                run(t_hbms[2], s_hbm, o_hbms[2], sid * (E // NC // NS),
                    E // NC // NS, buf_b, gsb, wsb)

        @pl.when(cid == 1)
        def _():
            run(t_hbms[1], d_hbm, o_hbms[1], sid * (E // NS), E // NS,
                buf_a, gsa, wsa)
            if nb:
                run(t_hbms[2], s_hbm, o_hbms[2],
                    E // NC + sid * (E // NC // NS), E // NC // NS,
                    buf_b, gsb, wsb)

    return k(src_r, dst_r, *tabs)


def _sc_scatter(msg, sigma, dst_r):
    RPW = 256
    HALF = 128
    NPAD = 10240

    out_type = [jax.ShapeDtypeStruct((N, D), jnp.float32)] * 2
    scratch = [pltpu.VMEM_SHARED((NPAD, D), jnp.float32),
               pltpu.VMEM((HALF, C), jnp.int32),
               pltpu.VMEM((C, D), jnp.float32),
               pltpu.VMEM((C, D), jnp.float32),
               pltpu.SemaphoreType.DMA,
               pltpu.SemaphoreType.DMA]

    @functools.partial(pl.kernel, out_type=out_type, mesh=_mesh(),
                       scratch_types=scratch)
    def k(m_hbm, s_hbm, d_hbm, num_hbm, den_hbm,
          acc, ibuf, v0, v1, sem0, sem1):
        cid = lax.axis_index("c")
        sid = lax.axis_index("s")
        last = sid == NS - 1

        @pl.loop(0, C)
        def _(r):
            @pl.loop(0, D, step=16)
            def _(cc):
                v0[r, pl.ds(cc, 16)] = jnp.zeros((16,), jnp.float32)

        @pl.loop(0, 8)
        def _(kz):
            pltpu.sync_copy(v0, acc.at[pl.ds(sid * 640 + kz * C, C)])

        plsc.subcore_barrier()

        def scat(data_hbm, out_hbm):
            for half in range(2):
                hrow0 = sid * RPW + half * HALF
                nblk = jnp.where(last, 128 if half == 0 else 32, HALF)

                @pl.when(last)
                def _():
                    nr = 128 if half == 0 else 32
                    pltpu.sync_copy(d_hbm.at[pl.ds(hrow0, nr)],
                                    ibuf.at[pl.ds(0, nr)])

                @pl.when(jnp.logical_not(last))
                def _():
                    pltpu.sync_copy(d_hbm.at[pl.ds(hrow0, HALF)], ibuf)

                def fire(b, buf, sem):
                    pltpu.async_copy(
                        data_hbm.at[pl.ds((hrow0 + b) * C, C)], buf, sem)

                def drain_scat(b, buf, sem):
                    pltpu.make_async_copy(data_hbm.at[pl.ds(0, C)], buf,
                                          sem).wait()
                    pltpu.sync_copy(buf, acc.at[ibuf.at[b]], add=True)

                fire(0, v0, sem0)

                @pl.loop(0, HALF // 2)
                def _(g):
                    b0 = 2 * g
                    b1 = b0 + 1

                    @pl.when(b1 < nblk)
                    def _():
                        fire(b1, v1, sem1)

                    @pl.when(b0 < nblk)
                    def _():
                        drain_scat(b0, v0, sem0)

                    @pl.when(b0 + 2 < nblk)
                    def _():
                        fire(b0 + 2, v0, sem0)

                    @pl.when(b1 < nblk)
                    def _():
                        drain_scat(b1, v1, sem1)

            plsc.subcore_barrier()

            ncp = jnp.where(last, 5, 8)

            @pl.loop(0, 8)
            def _(kz):
                @pl.when(kz < ncp)
                def _():
                    pltpu.sync_copy(acc.at[pl.ds(sid * 640 + kz * C, C)],
                                    out_hbm.at[pl.ds(sid * 640 + kz * C, C)])

        @pl.when(cid == 0)
        def _():
            scat(m_hbm, num_hbm)

        @pl.when(cid == 1)
        def _():
            scat(s_hbm, den_hbm)

    return k(msg, sigma, dst_r)


def kernel(x, edge_index, overlap_similarity, overlap_length, W_enc, b_enc,
           WA, bA, WB, bB, WC, bC, WD, bD, WE, bE, bn_h_gamma, bn_h_beta,
           bn_e_gamma, bn_e_beta, W1, b1, W2, b2):
    src = edge_index[0]
    dst = edge_index[1]
    dst_r = dst.reshape(NROW, C)
    sim2 = overlap_similarity.reshape(E, 1)
    ol2 = overlap_length.astype(jnp.float32).reshape(E, 1)

    olst = _ol_stats(overlap_length.astype(jnp.float32).reshape(E // 128, 128))
    e = _encode(sim2, ol2, olst, W_enc, b_enc)
    h = x

    L = WA.shape[0]
    for i in range(L):
        Ah, Bh, Dh, Eh = _matmul_multi(
            h, [WA[i], WB[i], WD[i], WE[i]], [bA[i], bB[i], bD[i], bE[i]])
        Ds, Ed, Bs = _sc_gather(Dh, Eh, Bh, src, dst)
        e_hat, stats = _edge_hat(e, Ds, Ed, WC[i], bC[i])
        stats = _edge_var(e_hat, stats)
        e, sigma, msg = _edge_update(e, e_hat, stats, Bs,
                                     bn_e_gamma[i], bn_e_beta[i])
        num, den = _sc_scatter(msg, sigma, dst_r)
        h = _node_update(h, Ah, num, den, bn_h_gamma[i], bn_h_beta[i])

    zb = jnp.zeros((D,), jnp.float32)
    Ps, Pd = _matmul_multi(h, [W1[0:D], W1[D:2 * D]], [zb, zb])
    Hs, Hd = _sc_gather(Ps, Pd, None, src, dst)
    return _decode(Hs, Hd, e, W1[2 * D:3 * D], b1, W2, b2)
